# Initial kernel scaffold; baseline (speedup 1.0000x reference)
import jax
import jax.numpy as jnp
from jax.experimental import pallas as pl


def kernel(x, table):
    raise NotImplementedError("write your pallas kernel here")


# speedup vs baseline: 1.8694x; 1.8694x over previous
import functools

import jax
import jax.numpy as jnp
from jax import lax
from jax.experimental import pallas as pl
from jax.experimental.pallas import tpu as pltpu
from jax.experimental.pallas import tpu_sc as plsc

EMB = 128
B = 4096
L = 50

NC = 2
NS = 16
NW = NC * NS

ROWS = B * L
RPW = ROWS // NW
GATHER = 100
CHUNK = 2 * GATHER
NCH = RPW // CHUNK

_mesh = plsc.VectorSubcoreMesh(core_axis_name="c", subcore_axis_name="s")


@functools.partial(
    pl.kernel,
    mesh=_mesh,
    out_type=jax.ShapeDtypeStruct((ROWS, EMB), jnp.float32),
    scratch_types=[
        pltpu.VMEM((2 * NCH, GATHER), jnp.int32),
        pltpu.VMEM((CHUNK, EMB), jnp.float32),
        pltpu.SemaphoreType.DMA,
    ],
)
def _shifted_gather(idx_hbm, table_hbm, out_hbm, idx_v, buf_v, sem):
    wid = lax.axis_index("s") * NC + lax.axis_index("c")
    pltpu.sync_copy(idx_hbm.at[wid], idx_v)
    zeros16 = jnp.zeros((16,), jnp.float32)

    def body(j, carry):
        cp0 = pltpu.async_copy(
            table_hbm.at[idx_v.at[2 * j]], buf_v.at[pl.ds(0, GATHER)], sem
        )
        cp1 = pltpu.async_copy(
            table_hbm.at[idx_v.at[2 * j + 1]], buf_v.at[pl.ds(GATHER, GATHER)], sem
        )
        cp0.wait()
        cp1.wait()
        for r in range(L - 1, CHUNK, L):
            for k in range(EMB // 16):
                buf_v[r, pl.ds(k * 16, 16)] = zeros16
        base = wid * RPW + j * CHUNK
        pltpu.sync_copy(buf_v, out_hbm.at[pl.ds(base, CHUNK)])
        return carry

    lax.fori_loop(0, NCH, body, 0)


def kernel(x, table):
    idx = jnp.concatenate(
        [x[:, 1:], jnp.zeros((B, 1), dtype=x.dtype)], axis=1
    ).astype(jnp.int32)
    idx = idx.reshape(NW, 2 * NCH, GATHER)
    out = _shifted_gather(idx, table)
    return out.reshape(B, L, EMB)
